# baseline (speedup 1.0000x reference)
import jax
import jax.numpy as jnp
from jax import lax
from jax.experimental import pallas as pl
from jax.experimental.pallas import tpu as pltpu
from jax.experimental.pallas import tpu_sc as plsc

B = 4
H = 512
W = 512
NM = 2048
NU = 1024
NC = 2
NS = 16
TILES_PER_SAMPLE = (NC * NS) // B
M_PER_TILE = NM // TILES_PER_SAMPLE
U_PER_TILE = NU // TILES_PER_SAMPLE
PER_TILE = 2 * M_PER_TILE + 2 * U_PER_TILE
CHUNK = 128


def _tile_body(pred_hbm, tgt_hbm,
               pmb, pmd, tmb, tmd, pub, pud, tub, tud,
               out_hbm,
               cpmb, cpmd, cpub, cpud, ctmb, ctmd, ctub, ctud,
               pidx, tidx, vp, vt, part, sem):
    cid = lax.axis_index("c")
    sid = lax.axis_index("s")
    wid = cid * NS + sid
    b = wid // TILES_PER_SAMPLE
    q = wid % TILES_PER_SAMPLE
    mo = q * M_PER_TILE
    uo = q * U_PER_TILE

    segs = ((cpmb, pmb, mo, M_PER_TILE), (cpmd, pmd, mo, M_PER_TILE),
            (cpub, pub, uo, U_PER_TILE), (cpud, pud, uo, U_PER_TILE),
            (ctmb, tmb, mo, M_PER_TILE), (ctmd, tmd, mo, M_PER_TILE),
            (ctub, tub, uo, U_PER_TILE), (ctud, tud, uo, U_PER_TILE))
    copies = [pltpu.async_copy(src.at[b, pl.ds(2 * o, 2 * n)], buf, sem)
              for (buf, src, o, n) in segs]
    for c in copies:
        c.wait()

    boff = b * (H * W)
    evens = lax.iota(jnp.int32, 16) * 2
    for (dst, bufs) in ((pidx, (cpmb, cpmd, cpub, cpud)),
                        (tidx, (ctmb, ctmd, ctub, ctud))):
        off = 0
        for buf in bufs:
            n = buf.shape[0] // 2
            for i in range(n // 16):
                yi = evens + (i * 32)
                ys = plsc.load_gather(buf, [yi])
                xs = plsc.load_gather(buf, [yi + 1])
                dst[pl.ds(off + i * 16, 16)] = boff + ys * W + xs
            off += n

    gathers = []
    for c in range(PER_TILE // CHUNK):
        o = c * CHUNK
        gathers.append(pltpu.async_copy(
            pred_hbm.at[pidx.at[pl.ds(o, CHUNK)]], vp.at[pl.ds(o, CHUNK)], sem))
        gathers.append(pltpu.async_copy(
            tgt_hbm.at[tidx.at[pl.ds(o, CHUNK)]], vt.at[pl.ds(o, CHUNK)], sem))
    for g in gathers:
        g.wait()

    acc_m = jnp.zeros((16,), jnp.float32)
    for i in range(2 * M_PER_TILE // 16):
        o = i * 16
        d = vp[pl.ds(o, 16)] - vt[pl.ds(o, 16)]
        acc_m = acc_m + d * d
    acc_u = jnp.zeros((16,), jnp.float32)
    for i in range(U_PER_TILE // 16):
        o = 2 * M_PER_TILE + i * 16
        d = vp[pl.ds(o, 16)] - vp[pl.ds(o + U_PER_TILE, 16)]
        e = vt[pl.ds(o, 16)] - vt[pl.ds(o + U_PER_TILE, 16)]
        acc_u = acc_u + d * d + e * e
    part[...] = (acc_m * 2.0 + acc_u) * (1.0 / B)
    pltpu.sync_copy(part, out_hbm.at[wid])


@jax.jit
def kernel(input, target, pred_matched_birth, pred_matched_death,
           tgt_matched_birth, tgt_matched_death,
           pred_unmatched_birth, pred_unmatched_death,
           tgt_unmatched_birth, tgt_unmatched_death):
    pred_flat = input.reshape(B * H * W)
    tgt_flat = target.reshape(B * H * W)

    mesh = plsc.VectorSubcoreMesh(core_axis_name="c", subcore_axis_name="s")
    run = pl.kernel(
        _tile_body,
        out_type=jax.ShapeDtypeStruct((NC * NS, 16), jnp.float32),
        mesh=mesh,
        compiler_params=pltpu.CompilerParams(needs_layout_passes=False),
        scratch_types=[
            pltpu.VMEM((2 * M_PER_TILE,), jnp.int32),
            pltpu.VMEM((2 * M_PER_TILE,), jnp.int32),
            pltpu.VMEM((2 * U_PER_TILE,), jnp.int32),
            pltpu.VMEM((2 * U_PER_TILE,), jnp.int32),
            pltpu.VMEM((2 * M_PER_TILE,), jnp.int32),
            pltpu.VMEM((2 * M_PER_TILE,), jnp.int32),
            pltpu.VMEM((2 * U_PER_TILE,), jnp.int32),
            pltpu.VMEM((2 * U_PER_TILE,), jnp.int32),
            pltpu.VMEM((PER_TILE,), jnp.int32),
            pltpu.VMEM((PER_TILE,), jnp.int32),
            pltpu.VMEM((PER_TILE,), jnp.float32),
            pltpu.VMEM((PER_TILE,), jnp.float32),
            pltpu.VMEM((16,), jnp.float32),
            pltpu.SemaphoreType.DMA,
        ],
    )
    parts = run(pred_flat, tgt_flat,
                pred_matched_birth.reshape(B, 2 * NM),
                pred_matched_death.reshape(B, 2 * NM),
                tgt_matched_birth.reshape(B, 2 * NM),
                tgt_matched_death.reshape(B, 2 * NM),
                pred_unmatched_birth.reshape(B, 2 * NU),
                pred_unmatched_death.reshape(B, 2 * NU),
                tgt_unmatched_birth.reshape(B, 2 * NU),
                tgt_unmatched_death.reshape(B, 2 * NU))
    return jnp.sum(parts)


# speedup vs baseline: 1.0202x; 1.0202x over previous
import jax
import jax.numpy as jnp
from jax import lax
from jax.experimental import pallas as pl
from jax.experimental.pallas import tpu as pltpu
from jax.experimental.pallas import tpu_sc as plsc

B = 4
H = 512
W = 512
NM = 2048
NU = 1024
NC = 2
NS = 16
TILES_PER_SAMPLE = (NC * NS) // B
M_PER_TILE = NM // TILES_PER_SAMPLE
U_PER_TILE = NU // TILES_PER_SAMPLE
PER_TILE = 2 * M_PER_TILE + 2 * U_PER_TILE
CHUNK = 128


def _dg(v, idx):
    return lax.gather(
        v, idx[:, None],
        lax.GatherDimensionNumbers(offset_dims=(), collapsed_slice_dims=(0,),
                                   start_index_map=(0,)),
        slice_sizes=(1,),
        mode=lax.GatherScatterMode.PROMISE_IN_BOUNDS)


def _deinterleave_linearize(cbuf, dst, off, n, boff):
    lanes = lax.iota(jnp.int32, 16)
    low = lanes < 8
    perm_y = (lanes * 2) % 16
    perm_x = (lanes * 2 + 1) % 16
    for i in range(n // 16):
        a = cbuf[pl.ds(32 * i, 16)]
        b = cbuf[pl.ds(32 * i + 16, 16)]
        ys = jnp.where(low, _dg(a, perm_y), _dg(b, perm_y))
        xs = jnp.where(low, _dg(a, perm_x), _dg(b, perm_x))
        dst[pl.ds(off + 16 * i, 16)] = boff + ys * W + xs


def _tile_body(pred_hbm, tgt_hbm,
               pmb, pmd, tmb, tmd, pub, pud, tub, tud,
               out_hbm,
               cpm, cpu_, ctm, ctu, pidx, tidx, vp, vt, part, sem):
    cid = lax.axis_index("c")
    sid = lax.axis_index("s")
    wid = cid * NS + sid
    b = wid // TILES_PER_SAMPLE
    q = wid % TILES_PER_SAMPLE
    mo = 2 * q * M_PER_TILE
    uo = 2 * q * U_PER_TILE

    ml = 2 * M_PER_TILE
    ul = 2 * U_PER_TILE
    segs = ((pmb, mo, cpm, 0, ml), (pmd, mo, cpm, ml, ml),
            (pub, uo, cpu_, 0, ul), (pud, uo, cpu_, ul, ul),
            (tmb, mo, ctm, 0, ml), (tmd, mo, ctm, ml, ml),
            (tub, uo, ctu, 0, ul), (tud, uo, ctu, ul, ul))
    copies = [pltpu.async_copy(src.at[b, pl.ds(o, n)],
                               buf.at[pl.ds(do, n)], sem)
              for (src, o, buf, do, n) in segs]
    for c in copies:
        c.wait()

    boff = b * (H * W)
    _deinterleave_linearize(cpm, pidx, 0, 2 * M_PER_TILE, boff)
    _deinterleave_linearize(cpu_, pidx, 2 * M_PER_TILE, 2 * U_PER_TILE, boff)
    _deinterleave_linearize(ctm, tidx, 0, 2 * M_PER_TILE, boff)
    _deinterleave_linearize(ctu, tidx, 2 * M_PER_TILE, 2 * U_PER_TILE, boff)

    gathers = []
    for c in range(PER_TILE // CHUNK):
        o = c * CHUNK
        gathers.append(pltpu.async_copy(
            pred_hbm.at[pidx.at[pl.ds(o, CHUNK)]], vp.at[pl.ds(o, CHUNK)], sem))
        gathers.append(pltpu.async_copy(
            tgt_hbm.at[tidx.at[pl.ds(o, CHUNK)]], vt.at[pl.ds(o, CHUNK)], sem))
    for g in gathers:
        g.wait()

    acc_m = jnp.zeros((16,), jnp.float32)
    for i in range(2 * M_PER_TILE // 16):
        o = i * 16
        d = vp[pl.ds(o, 16)] - vt[pl.ds(o, 16)]
        acc_m = acc_m + d * d
    acc_u = jnp.zeros((16,), jnp.float32)
    for i in range(U_PER_TILE // 16):
        o = 2 * M_PER_TILE + i * 16
        d = vp[pl.ds(o, 16)] - vp[pl.ds(o + U_PER_TILE, 16)]
        e = vt[pl.ds(o, 16)] - vt[pl.ds(o + U_PER_TILE, 16)]
        acc_u = acc_u + d * d + e * e
    part[...] = (acc_m * 2.0 + acc_u) * (1.0 / B)
    pltpu.sync_copy(part, out_hbm.at[wid])


@jax.jit
def kernel(input, target, pred_matched_birth, pred_matched_death,
           tgt_matched_birth, tgt_matched_death,
           pred_unmatched_birth, pred_unmatched_death,
           tgt_unmatched_birth, tgt_unmatched_death):
    pred_flat = input.reshape(B * H * W)
    tgt_flat = target.reshape(B * H * W)
    coords = [c.reshape(c.shape[0], -1) for c in (
        pred_matched_birth, pred_matched_death,
        tgt_matched_birth, tgt_matched_death,
        pred_unmatched_birth, pred_unmatched_death,
        tgt_unmatched_birth, tgt_unmatched_death)]

    mesh = plsc.VectorSubcoreMesh(core_axis_name="c", subcore_axis_name="s")
    run = pl.kernel(
        _tile_body,
        out_type=jax.ShapeDtypeStruct((NC * NS, 16), jnp.float32),
        mesh=mesh,
        scratch_types=[
            pltpu.VMEM((4 * M_PER_TILE,), jnp.int32),
            pltpu.VMEM((4 * U_PER_TILE,), jnp.int32),
            pltpu.VMEM((4 * M_PER_TILE,), jnp.int32),
            pltpu.VMEM((4 * U_PER_TILE,), jnp.int32),
            pltpu.VMEM((PER_TILE,), jnp.int32),
            pltpu.VMEM((PER_TILE,), jnp.int32),
            pltpu.VMEM((PER_TILE,), jnp.float32),
            pltpu.VMEM((PER_TILE,), jnp.float32),
            pltpu.VMEM((16,), jnp.float32),
            pltpu.SemaphoreType.DMA,
        ],
    )
    parts = run(pred_flat, tgt_flat, *coords)
    return jnp.sum(parts)


# speedup vs baseline: 1.4971x; 1.4675x over previous
import jax
import jax.numpy as jnp
from jax import lax
from jax.experimental import pallas as pl
from jax.experimental.pallas import tpu as pltpu
from jax.experimental.pallas import tpu_sc as plsc

B = 4
H = 512
W = 512
NM = 2048
NU = 1024
NC = 2
NS = 16
TILES_PER_SAMPLE = (NC * NS) // B
M_PER_TILE = NM // TILES_PER_SAMPLE
U_PER_TILE = NU // TILES_PER_SAMPLE
PER_TILE = 2 * M_PER_TILE + 2 * U_PER_TILE
ROW = 4 * PER_TILE
CHUNK = 128


def _tile_body(pred_hbm, tgt_hbm, coords_hbm, out_hbm,
               cbuf, pidx, tidx, vp, vt, part, sem):
    cid = lax.axis_index("c")
    sid = lax.axis_index("s")
    wid = cid * NS + sid
    b = wid // TILES_PER_SAMPLE
    q = wid % TILES_PER_SAMPLE

    mo = q * M_PER_TILE
    uo = q * U_PER_TILE
    ml = M_PER_TILE
    ul = U_PER_TILE
    segs = ((0 * NM + mo, 0, ml), (1 * NM + mo, ml, ml),
            (4 * NM + 0 * NU + uo, 2 * ml, ul),
            (4 * NM + 1 * NU + uo, 2 * ml + ul, ul))
    tsegs = ((2 * NM + mo, 0, ml), (3 * NM + mo, ml, ml),
             (4 * NM + 2 * NU + uo, 2 * ml, ul),
             (4 * NM + 3 * NU + uo, 2 * ml + ul, ul))
    copies = []
    for (base, group) in ((0, segs), (2 * PER_TILE, tsegs)):
        for d in (0, 1):
            for (so, do, n) in group:
                copies.append(pltpu.async_copy(
                    coords_hbm.at[b, d, pl.ds(so, n)],
                    cbuf.at[pl.ds(base + d * PER_TILE + do, n)], sem))
    for c in copies:
        c.wait()

    boff = b * (H * W)
    for i in range(PER_TILE // 16):
        o = i * 16
        pidx[pl.ds(o, 16)] = (boff + cbuf[pl.ds(o, 16)] * W
                              + cbuf[pl.ds(PER_TILE + o, 16)])
        tidx[pl.ds(o, 16)] = (boff + cbuf[pl.ds(2 * PER_TILE + o, 16)] * W
                              + cbuf[pl.ds(3 * PER_TILE + o, 16)])

    gathers = []
    for c in range(PER_TILE // CHUNK):
        o = c * CHUNK
        gathers.append(pltpu.async_copy(
            pred_hbm.at[pidx.at[pl.ds(o, CHUNK)]], vp.at[pl.ds(o, CHUNK)], sem))
        gathers.append(pltpu.async_copy(
            tgt_hbm.at[tidx.at[pl.ds(o, CHUNK)]], vt.at[pl.ds(o, CHUNK)], sem))
    for g in gathers:
        g.wait()

    acc_m = jnp.zeros((16,), jnp.float32)
    for i in range(2 * M_PER_TILE // 16):
        o = i * 16
        d = vp[pl.ds(o, 16)] - vt[pl.ds(o, 16)]
        acc_m = acc_m + d * d
    acc_u = jnp.zeros((16,), jnp.float32)
    for i in range(U_PER_TILE // 16):
        o = 2 * M_PER_TILE + i * 16
        d = vp[pl.ds(o, 16)] - vp[pl.ds(o + U_PER_TILE, 16)]
        e = vt[pl.ds(o, 16)] - vt[pl.ds(o + U_PER_TILE, 16)]
        acc_u = acc_u + d * d + e * e
    part[...] = (acc_m * 2.0 + acc_u) * (1.0 / B)
    pltpu.sync_copy(part, out_hbm.at[wid])


@jax.jit
def kernel(input, target, pred_matched_birth, pred_matched_death,
           tgt_matched_birth, tgt_matched_death,
           pred_unmatched_birth, pred_unmatched_death,
           tgt_unmatched_birth, tgt_unmatched_death):
    pred_flat = input.reshape(B * H * W)
    tgt_flat = target.reshape(B * H * W)

    coords = jnp.concatenate(
        [pred_matched_birth, pred_matched_death,
         tgt_matched_birth, tgt_matched_death,
         pred_unmatched_birth, pred_unmatched_death,
         tgt_unmatched_birth, tgt_unmatched_death],
        axis=1).transpose(0, 2, 1)

    mesh = plsc.VectorSubcoreMesh(core_axis_name="c", subcore_axis_name="s")
    run = pl.kernel(
        _tile_body,
        out_type=jax.ShapeDtypeStruct((NC * NS, 16), jnp.float32),
        mesh=mesh,
        scratch_types=[
            pltpu.VMEM((ROW,), jnp.int32),
            pltpu.VMEM((PER_TILE,), jnp.int32),
            pltpu.VMEM((PER_TILE,), jnp.int32),
            pltpu.VMEM((PER_TILE,), jnp.float32),
            pltpu.VMEM((PER_TILE,), jnp.float32),
            pltpu.VMEM((16,), jnp.float32),
            pltpu.SemaphoreType.DMA,
        ],
    )
    parts = run(pred_flat, tgt_flat, coords)
    return jnp.sum(parts)
